# baseline (speedup 1.0000x reference)
import jax
import jax.numpy as jnp
from jax import lax
from jax.experimental import pallas as pl
from jax.experimental.pallas import tpu as pltpu
from jax.experimental.pallas import tpu_sc as plsc

_EMBED_DIM = 32
_BATCH = 16384

_info = plsc.get_sparse_core_info()
_NC, _NS, _NL = _info.num_cores, _info.num_subcores, _info.num_lanes
_NW = _NC * _NS
_BPW = _BATCH // _NW
_CHUNK = 128
_NCHUNK = _BPW // _CHUNK


def _emb_kernel(tidx_hbm, sidx_hbm, ttab_hbm, stab_hbm, out_hbm,
                tidx_v, sidx_v, tdst_v, sdst_v, trows_v, srows_v,
                sem_g, sem_sc):
    wid = lax.axis_index("s") * _NC + lax.axis_index("c")
    base = wid * _BPW

    pltpu.sync_copy(tidx_hbm.at[pl.ds(wid * _NCHUNK, _NCHUNK)], tidx_v)
    pltpu.sync_copy(sidx_hbm.at[pl.ds(wid * _NCHUNK, _NCHUNK)], sidx_v)

    lane = lax.iota(jnp.int32, _NL)
    for j in range(_NCHUNK):
        for t in range(_CHUNK // _NL):
            off = 2 * (base + j * _CHUNK + t * _NL)
            tdst_v[j, pl.ds(t * _NL, _NL)] = off + 2 * lane
            sdst_v[j, pl.ds(t * _NL, _NL)] = off + 2 * lane + 1

    copies = []
    for j in range(_NCHUNK):
        copies.append(pltpu.async_copy(
            ttab_hbm.at[tidx_v.at[j]],
            trows_v.at[pl.ds(j * _CHUNK, _CHUNK)], sem_g))
        copies.append(pltpu.async_copy(
            stab_hbm.at[sidx_v.at[j]],
            srows_v.at[pl.ds(j * _CHUNK, _CHUNK)], sem_g))
    for c in copies:
        c.wait()

    copies = []
    for j in range(_NCHUNK):
        copies.append(pltpu.async_copy(
            trows_v.at[pl.ds(j * _CHUNK, _CHUNK)],
            out_hbm.at[tdst_v.at[j]], sem_sc))
        copies.append(pltpu.async_copy(
            srows_v.at[pl.ds(j * _CHUNK, _CHUNK)],
            out_hbm.at[sdst_v.at[j]], sem_sc))
    for c in copies:
        c.wait()


@jax.jit
def _run(tweet, sentiment, tweet_table, sentiment_table):
    mesh = plsc.VectorSubcoreMesh(core_axis_name="c", subcore_axis_name="s")
    out = pl.kernel(
        _emb_kernel,
        out_type=jax.ShapeDtypeStruct((2 * _BATCH, _EMBED_DIM), jnp.float32),
        mesh=mesh,
        compiler_params=pltpu.CompilerParams(use_tc_tiling_on_sc=False),
        scratch_types=[
            pltpu.VMEM((_NCHUNK, _CHUNK), jnp.int32),
            pltpu.VMEM((_NCHUNK, _CHUNK), jnp.int32),
            pltpu.VMEM((_NCHUNK, _CHUNK), jnp.int32),
            pltpu.VMEM((_NCHUNK, _CHUNK), jnp.int32),
            pltpu.VMEM((_BPW, _EMBED_DIM), jnp.float32),
            pltpu.VMEM((_BPW, _EMBED_DIM), jnp.float32),
            pltpu.SemaphoreType.DMA,
            pltpu.SemaphoreType.DMA,
        ],
    )(tweet.reshape(_BATCH // _CHUNK, _CHUNK),
      sentiment.reshape(_BATCH // _CHUNK, _CHUNK),
      tweet_table, sentiment_table)
    return out.reshape(_BATCH, 2 * _EMBED_DIM)


def kernel(tweet, sentiment, tweet_table, sentiment_table):
    return _run(tweet, sentiment, tweet_table, sentiment_table)
